# baseline (speedup 1.0000x reference)
import functools

import jax
import jax.numpy as jnp
from jax import lax
from jax.experimental import pallas as pl
from jax.experimental.pallas import tpu as pltpu
from jax.experimental.pallas import tpu_sc as plsc

N_NODES = 10000
N_EDGES = 320000
D_IN = 128
D_OUT = 128

NC = 2
NS = 16
NW = NC * NS
CHUNK = 128
NCHUNK = 80
EPW = NCHUNK * CHUNK
E_PAD = NW * EPW
ROWS_PT = 624
ROWS_LAST = N_NODES - (NS - 1) * ROWS_PT
LANES = 16
VPR = D_OUT // LANES


def _mm_body(x_ref, w_ref, o_ref):
    o_ref[...] = jnp.dot(x_ref[...], w_ref[...],
                         preferred_element_type=jnp.float32)


def _matmul(x, W):
    grid = 10
    rows = N_NODES // grid
    return pl.pallas_call(
        _mm_body,
        grid=(grid,),
        in_specs=[
            pl.BlockSpec((rows, D_IN), lambda i: (i, 0)),
            pl.BlockSpec((D_IN, D_OUT), lambda i: (0, 0)),
        ],
        out_specs=pl.BlockSpec((rows, D_OUT), lambda i: (i, 0)),
        out_shape=jax.ShapeDtypeStruct((N_NODES, D_OUT), jnp.float32),
    )(x, W)


_mesh = plsc.VectorSubcoreMesh(core_axis_name="c", subcore_axis_name="s")


@functools.partial(
    pl.kernel,
    out_type=jax.ShapeDtypeStruct((NC, N_NODES, D_OUT), jnp.float32),
    mesh=_mesh,
    scratch_types=[
        pltpu.VMEM((2, CHUNK), jnp.int32),
        pltpu.VMEM((2, CHUNK), jnp.int32),
        pltpu.VMEM((2, CHUNK), jnp.float32),
        pltpu.VMEM((CHUNK, D_OUT), jnp.float32),
        pltpu.VMEM((CHUNK, D_OUT), jnp.float32),
        pltpu.VMEM_SHARED((N_NODES, D_OUT), jnp.float32),
        pltpu.SemaphoreType.DMA,
        pltpu.SemaphoreType.DMA,
        pltpu.SemaphoreType.DMA,
        pltpu.SemaphoreType.DMA,
    ],
)
def _sc_edges(src_hbm, dst_hbm, w_hbm, h_hbm, out_hbm,
              src_v, dst_v, w_v, rows0, rows1, acc_sh,
              gat0, gat1, isem0, isem1):
    cid = lax.axis_index("c")
    sid = lax.axis_index("s")
    wid = sid * NC + cid
    rows = (rows0, rows1)
    gat = (gat0, gat1)
    isem = (isem0, isem1)

    def _load_idx(i, p, sync):
        for hb, vb in ((src_hbm, src_v), (dst_hbm, dst_v), (w_hbm, w_v)):
            if sync:
                pltpu.sync_copy(hb.at[wid, i], vb.at[p])
            else:
                pltpu.async_copy(hb.at[wid, i], vb.at[p], isem[p])

    def _wait_idx(i, p):
        for hb, vb in ((src_hbm, src_v), (dst_hbm, dst_v), (w_hbm, w_v)):
            pltpu.make_async_copy(hb.at[wid, i], vb.at[p], isem[p]).wait()

    zvec = jnp.zeros((LANES,), jnp.float32)

    def _zero_row(r, _):
        for j in range(VPR):
            rows0[r, pl.ds(j * LANES, LANES)] = zvec
        return 0

    lax.fori_loop(0, CHUNK, _zero_row, 0)
    row_base = pl.multiple_of(sid * ROWS_PT, 8)
    nfull = ROWS_PT // CHUNK
    rem = ROWS_PT - nfull * CHUNK
    rem_last = ROWS_LAST - nfull * CHUNK
    for z in range(nfull):
        pltpu.sync_copy(rows0,
                        acc_sh.at[pl.ds(row_base + z * CHUNK, CHUNK)])

    @pl.when(sid < NS - 1)
    def _zero_tail():
        pltpu.sync_copy(rows0.at[pl.ds(0, rem)],
                        acc_sh.at[pl.ds(row_base + nfull * CHUNK, rem)])

    @pl.when(sid == NS - 1)
    def _zero_tail_last():
        pltpu.sync_copy(rows0.at[pl.ds(0, rem_last)],
                        acc_sh.at[pl.ds((NS - 1) * ROWS_PT + nfull * CHUNK,
                                        rem_last)])

    def _gather(i, b):
        pltpu.async_copy(h_hbm.at[src_v.at[b]], rows[b], gat[b])

    def _wait_gather(b):
        pltpu.make_async_copy(h_hbm.at[src_v.at[b]], rows[b], gat[b]).wait()

    def _scale(b):
        def _group(g, _):
            wv = w_v[b, pl.ds(g * LANES, LANES)]
            for t in range(LANES):
                e = g * LANES + t
                w = wv[t]
                for j in range(VPR):
                    sl = pl.ds(j * LANES, LANES)
                    rows[b][e, sl] = rows[b][e, sl] * w
            return 0

        lax.fori_loop(0, CHUNK // LANES, _group, 0)

    _load_idx(0, 0, True)
    _load_idx(1, 1, False)
    _gather(0, 0)

    plsc.subcore_barrier()

    def _slot(i, b):
        nb = (b + 1) % 2
        @pl.when(i + 1 < NCHUNK)
        def _ahead():
            _wait_idx(i + 1, nb)
            _gather(i + 1, nb)

        _wait_gather(b)
        _scale(b)
        pltpu.sync_copy(rows[b], acc_sh.at[dst_v.at[b]], add=True)

        @pl.when(i + 2 < NCHUNK)
        def _prefetch():
            _load_idx(i + 2, b, False)

    def _pair(t, _):
        _slot(2 * t, 0)
        _slot(2 * t + 1, 1)
        return 0

    lax.fori_loop(0, NCHUNK // 2, _pair, 0)
    plsc.subcore_barrier()

    @pl.when(sid < NS - 1)
    def _wb_main():
        pltpu.sync_copy(acc_sh.at[pl.ds(row_base, ROWS_PT)],
                        out_hbm.at[cid, pl.ds(row_base, ROWS_PT)])

    @pl.when(sid == NS - 1)
    def _wb_last():
        last = (NS - 1) * ROWS_PT
        pltpu.sync_copy(acc_sh.at[pl.ds(last, ROWS_LAST)],
                        out_hbm.at[cid, pl.ds(last, ROWS_LAST)])


def _add_body(a_ref, b_ref, o_ref):
    o_ref[...] = a_ref[...] + b_ref[...]


def _combine(p0, p1):
    grid = 10
    rows = N_NODES // grid
    return pl.pallas_call(
        _add_body,
        grid=(grid,),
        in_specs=[
            pl.BlockSpec((rows, D_OUT), lambda i: (i, 0)),
            pl.BlockSpec((rows, D_OUT), lambda i: (i, 0)),
        ],
        out_specs=pl.BlockSpec((rows, D_OUT), lambda i: (i, 0)),
        out_shape=jax.ShapeDtypeStruct((N_NODES, D_OUT), jnp.float32),
    )(p0, p1)


def kernel(x, edge_index, edge_weight, W):
    edge_index = edge_index.astype(jnp.int32)
    pad = E_PAD - N_EDGES
    src = jnp.concatenate(
        [edge_index[0], jnp.zeros((pad,), jnp.int32)]).reshape(
            NW, NCHUNK, CHUNK)
    dst = jnp.concatenate(
        [edge_index[1], jnp.zeros((pad,), jnp.int32)]).reshape(
            NW, NCHUNK, CHUNK)
    ew = jnp.concatenate(
        [edge_weight, jnp.zeros((pad,), jnp.float32)]).reshape(
            NW, NCHUNK, CHUNK)
    h = _matmul(x, W)
    partials = _sc_edges(src, dst, ew, h)
    return _combine(partials[0], partials[1])
